# Initial kernel scaffold; baseline (speedup 1.0000x reference)
import jax
import jax.numpy as jnp
from jax.experimental import pallas as pl


def kernel(x, edge_index, W, b, Wm, bm, Wa):
    raise NotImplementedError("write your pallas kernel here")


# speedup vs baseline: 10.4106x; 10.4106x over previous
import functools

import jax
import jax.numpy as jnp
from jax import lax
from jax.experimental import pallas as pl
from jax.experimental.pallas import tpu as pltpu
from jax.experimental.pallas import tpu_sc as plsc

_N = 10000
_NPAD = 10240
_E = 320000
_D = 128
_NC = 2
_NS = 16
_NW = _NC * _NS
_EPW = _E // _NW
_G = 16
_NG = _EPW // _G
_CH = _NPAD // _NS
_PB = 512


def _pre_body(x_ref, w_ref, b_ref, wm_ref, bm_ref, wa_ref, h_ref, s1_ref, s2_ref):
    x = x_ref[...]
    h_raw = lax.dot_general(x, w_ref[...], (((1,), (1,)), ((), ())),
                            preferred_element_type=jnp.float32) + b_ref[...]
    nrm = jnp.sqrt(jnp.sum(h_raw * h_raw, axis=1, keepdims=True))
    h_dir = h_raw / jnp.maximum(nrm, 1e-12)
    mag_lin = jnp.sum(x * wm_ref[...], axis=1, keepdims=True) + bm_ref[...]
    mag = jax.nn.sigmoid(mag_lin) * 1.5
    h = h_dir * mag
    h_ref[...] = h
    wa = wa_ref[...]
    s1_ref[...] = jnp.sum(h * wa[0:1, :], axis=1, keepdims=True)
    s2_ref[...] = jnp.sum(h * wa[1:2, :], axis=1, keepdims=True)


_pre_call = pl.pallas_call(
    _pre_body,
    grid=(_NPAD // _PB,),
    in_specs=[
        pl.BlockSpec((_PB, _D), lambda i: (i, 0)),
        pl.BlockSpec((_D, _D), lambda i: (0, 0)),
        pl.BlockSpec((1, _D), lambda i: (0, 0)),
        pl.BlockSpec((1, _D), lambda i: (0, 0)),
        pl.BlockSpec((1, 1), lambda i: (0, 0)),
        pl.BlockSpec((2, _D), lambda i: (0, 0)),
    ],
    out_specs=[
        pl.BlockSpec((_PB, _D), lambda i: (i, 0)),
        pl.BlockSpec((_PB, 1), lambda i: (i, 0)),
        pl.BlockSpec((_PB, 1), lambda i: (i, 0)),
    ],
    out_shape=[
        jax.ShapeDtypeStruct((_NPAD, _D), jnp.float32),
        jax.ShapeDtypeStruct((_NPAD, 1), jnp.float32),
        jax.ShapeDtypeStruct((_NPAD, 1), jnp.float32),
    ],
)


def _post_body(h_ref, a0_ref, a1_ref, o_ref):
    hc = h_ref[...] + a0_ref[...] + a1_ref[...]
    un = jnp.sqrt(jnp.sum(hc * hc, axis=1, keepdims=True))
    unc = jnp.maximum(un, 1e-15)
    hp = jnp.tanh(unc) * hc / unc
    nn = jnp.sqrt(jnp.sum(hp * hp, axis=1, keepdims=True))
    o_ref[...] = jnp.where(nn > 0.95, hp * (0.95 / (nn + 1e-8)), hp)


_post_call = pl.pallas_call(
    _post_body,
    grid=(_NPAD // _PB,),
    in_specs=[pl.BlockSpec((_PB, _D), lambda i: (i, 0))] * 3,
    out_specs=pl.BlockSpec((_PB, _D), lambda i: (i, 0)),
    out_shape=jax.ShapeDtypeStruct((_NPAD, _D), jnp.float32),
)


_mesh = plsc.VectorSubcoreMesh(core_axis_name="c", subcore_axis_name="s")


@functools.partial(
    pl.kernel,
    out_type=[
        jax.ShapeDtypeStruct((_E,), jnp.float32),
        jax.ShapeDtypeStruct((_NC, _NPAD), jnp.float32),
    ],
    mesh=_mesh,
    compiler_params=pltpu.CompilerParams(needs_layout_passes=False),
    scratch_types=[
        pltpu.VMEM((_NPAD,), jnp.float32),
        pltpu.VMEM((_NPAD,), jnp.float32),
        pltpu.VMEM((_EPW,), jnp.int32),
        pltpu.VMEM((_EPW,), jnp.int32),
        pltpu.VMEM((_EPW,), jnp.float32),
        pltpu.VMEM((_NPAD,), jnp.float32),
        pltpu.VMEM((_NS, _CH), jnp.float32),
        pltpu.VMEM_SHARED((_NS, _NPAD), jnp.float32),
    ],
)
def _edge_phase_a(src_h, tgt_h, s1_h, s2_h, evals_h, sump_h,
                  s1_v, s2_v, src_v, tgt_v, ev_v, sum_v, red_v, shared):
    c = lax.axis_index("c")
    s = lax.axis_index("s")
    wid = s * _NC + c
    base = wid * _EPW
    pltpu.sync_copy(s1_h, s1_v)
    pltpu.sync_copy(s2_h, s2_v)
    pltpu.sync_copy(src_h.at[pl.ds(base, _EPW)], src_v)
    pltpu.sync_copy(tgt_h.at[pl.ds(base, _EPW)], tgt_v)

    zv = jnp.zeros((16,), jnp.float32)

    def _zero(i, carry):
        sum_v[pl.ds(i * 16, 16)] = zv
        return carry

    lax.fori_loop(0, _NPAD // 16, _zero, 0)

    def _edges(i, carry):
        o = i * 16
        si = src_v[pl.ds(o, 16)]
        ti = tgt_v[pl.ds(o, 16)]
        ev = jnp.exp(plsc.load_gather(s1_v, [si]) + plsc.load_gather(s2_v, [ti]))
        ev_v[pl.ds(o, 16)] = ev
        plsc.addupdate_scatter(sum_v, [ti], ev)
        return carry

    lax.fori_loop(0, _EPW // 16, _edges, 0)

    pltpu.sync_copy(ev_v, evals_h.at[pl.ds(base, _EPW)])
    pltpu.sync_copy(sum_v, shared.at[s])
    plsc.subcore_barrier()

    ch = s * _CH
    pltpu.sync_copy(shared.at[:, pl.ds(ch, _CH)], red_v)

    def _reduce(j, carry):
        acc = red_v[0, pl.ds(j * 16, 16)]
        for k in range(1, _NS):
            acc = acc + red_v[k, pl.ds(j * 16, 16)]
        sum_v[pl.ds(j * 16, 16)] = acc
        return carry

    lax.fori_loop(0, _CH // 16, _reduce, 0)
    pltpu.sync_copy(sum_v.at[pl.ds(0, _CH)], sump_h.at[c, pl.ds(ch, _CH)])


@functools.partial(
    pl.kernel,
    out_type=jax.ShapeDtypeStruct((_NC, _NPAD, _D), jnp.float32),
    mesh=_mesh,
    compiler_params=pltpu.CompilerParams(needs_layout_passes=False),
    scratch_types=[
        pltpu.VMEM((_EPW,), jnp.int32),
        pltpu.VMEM((_EPW,), jnp.int32),
        pltpu.VMEM((_EPW,), jnp.float32),
        pltpu.VMEM((_NPAD,), jnp.float32),
        pltpu.VMEM((_CH,), jnp.float32),
        pltpu.VMEM((_G, _D), jnp.float32),
        pltpu.VMEM((_G, _D), jnp.float32),
        pltpu.VMEM_SHARED((_NPAD, _D), jnp.float32),
        pltpu.SemaphoreType.DMA,
    ],
)
def _edge_phase_b(src_h, tgt_h, evals_h, sump_h, h_hbm, aggp_h,
                  src_v, tgt_v, w_v, recip_v, tmp_v, rows_v, scaled_v,
                  agg_s, sem):
    c = lax.axis_index("c")
    s = lax.axis_index("s")
    wid = s * _NC + c
    base = wid * _EPW
    pltpu.sync_copy(src_h.at[pl.ds(base, _EPW)], src_v)
    pltpu.sync_copy(tgt_h.at[pl.ds(base, _EPW)], tgt_v)
    pltpu.sync_copy(evals_h.at[pl.ds(base, _EPW)], w_v)
    pltpu.sync_copy(sump_h.at[0], recip_v)

    def _recip_chunk(cidx, carry):
        cb = cidx * _CH
        pltpu.sync_copy(sump_h.at[1, pl.ds(cb, _CH)], tmp_v)

        def _recip(i, inner):
            o = cb + i * 16
            recip_v[pl.ds(o, 16)] = 1.0 / (recip_v[pl.ds(o, 16)]
                                           + tmp_v[pl.ds(i * 16, 16)] + 1e-10)
            return inner

        lax.fori_loop(0, _CH // 16, _recip, 0)
        return carry

    lax.fori_loop(0, _NPAD // _CH, _recip_chunk, 0)

    def _weights(i, carry):
        o = i * 16
        ti = tgt_v[pl.ds(o, 16)]
        w_v[pl.ds(o, 16)] = w_v[pl.ds(o, 16)] * plsc.load_gather(recip_v, [ti])
        return carry

    lax.fori_loop(0, _EPW // 16, _weights, 0)

    zv = jnp.zeros((16,), jnp.float32)
    for j in range(_G):
        for k in range(_D // 16):
            scaled_v[j, pl.ds(k * 16, 16)] = zv

    def _zero(k, carry):
        pltpu.sync_copy(scaled_v, agg_s.at[pl.ds(s * _CH + k * _G, _G), :])
        return carry

    lax.fori_loop(0, _CH // _G, _zero, 0)
    plsc.subcore_barrier()

    def _groups(g, carry):
        o = g * _G
        si = src_v[pl.ds(o, _G)]
        pltpu.async_copy(h_hbm.at[si], rows_v, sem).wait()
        wv = w_v[pl.ds(o, _G)]
        for j in range(_G):
            wj = wv[j]
            for k in range(_D // 16):
                sl = pl.ds(k * 16, 16)
                scaled_v[j, sl] = rows_v[j, sl] * wj
        ti = tgt_v[pl.ds(o, _G)]
        pltpu.sync_copy(scaled_v, agg_s.at[ti], add=True)
        return carry

    lax.fori_loop(0, _NG, _groups, 0)
    plsc.subcore_barrier()

    def _out(k, carry):
        r = s * _CH + k * _G
        pltpu.sync_copy(agg_s.at[pl.ds(r, _G), :], rows_v)
        pltpu.sync_copy(rows_v, aggp_h.at[c, pl.ds(r, _G), :])
        return carry

    lax.fori_loop(0, _CH // _G, _out, 0)


def kernel(x, edge_index, W, b, Wm, bm, Wa):
    x_p = jnp.pad(x, ((0, _NPAD - _N), (0, 0)))
    h, s1c, s2c = _pre_call(x_p, W, b.reshape(1, _D), Wm, bm.reshape(1, 1),
                            Wa.reshape(2, _D))
    src = edge_index[0]
    tgt = edge_index[1]
    s1 = s1c.reshape(_NPAD)
    s2 = s2c.reshape(_NPAD)
    evals, sump = _edge_phase_a(src, tgt, s1, s2)
    aggp = _edge_phase_b(src, tgt, evals, sump, h)
    out = _post_call(h, aggp[0], aggp[1])
    return out[:_N]


# speedup vs baseline: 17.8986x; 1.7193x over previous
import functools

import jax
import jax.numpy as jnp
from jax import lax
from jax.experimental import pallas as pl
from jax.experimental.pallas import tpu as pltpu
from jax.experimental.pallas import tpu_sc as plsc

_N = 10000
_NPAD = 10240
_E = 320000
_D = 128
_NC = 2
_NS = 16
_NW = _NC * _NS
_EPW = 10016
_EPAD = _EPW * _NW
_G = 16
_NG = _EPW // _G
_NG2 = _NG // 2
_CH = _NPAD // _NS
_NAGG = 10112
_CHA = _NAGG // _NS
_PB = 512


def _pre_body(x_ref, w_ref, b_ref, wm_ref, bm_ref, wa_ref, h_ref, s1_ref, s2_ref):
    x = x_ref[...]
    h_raw = lax.dot_general(x, w_ref[...], (((1,), (1,)), ((), ())),
                            preferred_element_type=jnp.float32) + b_ref[...]
    nrm = jnp.sqrt(jnp.sum(h_raw * h_raw, axis=1, keepdims=True))
    h_dir = h_raw / jnp.maximum(nrm, 1e-12)
    mag_lin = jnp.sum(x * wm_ref[...], axis=1, keepdims=True) + bm_ref[...]
    mag = jax.nn.sigmoid(mag_lin) * 1.5
    h = h_dir * mag
    h_ref[...] = h
    wa = wa_ref[...]
    s1_ref[...] = jnp.sum(h * wa[0:1, :], axis=1, keepdims=True)
    s2_ref[...] = jnp.sum(h * wa[1:2, :], axis=1, keepdims=True)


_pre_call = pl.pallas_call(
    _pre_body,
    grid=(_NPAD // _PB,),
    in_specs=[
        pl.BlockSpec((_PB, _D), lambda i: (i, 0)),
        pl.BlockSpec((_D, _D), lambda i: (0, 0)),
        pl.BlockSpec((1, _D), lambda i: (0, 0)),
        pl.BlockSpec((1, _D), lambda i: (0, 0)),
        pl.BlockSpec((1, 1), lambda i: (0, 0)),
        pl.BlockSpec((2, _D), lambda i: (0, 0)),
    ],
    out_specs=[
        pl.BlockSpec((_PB, _D), lambda i: (i, 0)),
        pl.BlockSpec((_PB, 1), lambda i: (i, 0)),
        pl.BlockSpec((_PB, 1), lambda i: (i, 0)),
    ],
    out_shape=[
        jax.ShapeDtypeStruct((_NPAD, _D), jnp.float32),
        jax.ShapeDtypeStruct((_NPAD, 1), jnp.float32),
        jax.ShapeDtypeStruct((_NPAD, 1), jnp.float32),
    ],
)


def _post_body(h_ref, a0_ref, a1_ref, o_ref):
    hc = h_ref[...] + a0_ref[...] + a1_ref[...]
    un = jnp.sqrt(jnp.sum(hc * hc, axis=1, keepdims=True))
    unc = jnp.maximum(un, 1e-15)
    hp = jnp.tanh(unc) * hc / unc
    nn = jnp.sqrt(jnp.sum(hp * hp, axis=1, keepdims=True))
    o_ref[...] = jnp.where(nn > 0.95, hp * (0.95 / (nn + 1e-8)), hp)


_post_call = pl.pallas_call(
    _post_body,
    grid=(_NPAD // _PB,),
    in_specs=[pl.BlockSpec((_PB, _D), lambda i: (i, 0))] * 3,
    out_specs=pl.BlockSpec((_PB, _D), lambda i: (i, 0)),
    out_shape=jax.ShapeDtypeStruct((_NPAD, _D), jnp.float32),
)


_mesh = plsc.VectorSubcoreMesh(core_axis_name="c", subcore_axis_name="s")


@functools.partial(
    pl.kernel,
    out_type=[
        jax.ShapeDtypeStruct((_EPAD,), jnp.float32),
        jax.ShapeDtypeStruct((_NC, _NPAD), jnp.float32),
    ],
    mesh=_mesh,
    compiler_params=pltpu.CompilerParams(needs_layout_passes=False),
    scratch_types=[
        pltpu.VMEM((_NPAD,), jnp.float32),
        pltpu.VMEM((_NPAD,), jnp.float32),
        pltpu.VMEM((_EPW,), jnp.int32),
        pltpu.VMEM((_EPW,), jnp.int32),
        pltpu.VMEM((_EPW,), jnp.float32),
        pltpu.VMEM((_NPAD,), jnp.float32),
        pltpu.VMEM((_NS, _CH), jnp.float32),
        pltpu.VMEM_SHARED((_NS, _NPAD), jnp.float32),
    ],
)
def _edge_phase_a(src_h, tgt_h, s1_h, s2_h, evals_h, sump_h,
                  s1_v, s2_v, src_v, tgt_v, ev_v, sum_v, red_v, shared):
    c = lax.axis_index("c")
    s = lax.axis_index("s")
    wid = s * _NC + c
    base = wid * _EPW
    pltpu.sync_copy(s1_h, s1_v)
    pltpu.sync_copy(s2_h, s2_v)
    pltpu.sync_copy(src_h.at[pl.ds(base, _EPW)], src_v)
    pltpu.sync_copy(tgt_h.at[pl.ds(base, _EPW)], tgt_v)

    zv = jnp.zeros((16,), jnp.float32)

    def _zero(i, carry):
        sum_v[pl.ds(i * 16, 16)] = zv
        return carry

    lax.fori_loop(0, _NPAD // 16, _zero, 0)

    def _edges(i, carry):
        o = i * 16
        si = src_v[pl.ds(o, 16)]
        ti = tgt_v[pl.ds(o, 16)]
        ev = jnp.exp(plsc.load_gather(s1_v, [si]) + plsc.load_gather(s2_v, [ti]))
        ev_v[pl.ds(o, 16)] = ev
        plsc.addupdate_scatter(sum_v, [ti], ev)
        return carry

    lax.fori_loop(0, _EPW // 16, _edges, 0)

    pltpu.sync_copy(ev_v, evals_h.at[pl.ds(base, _EPW)])
    pltpu.sync_copy(sum_v, shared.at[s])
    plsc.subcore_barrier()

    ch = s * _CH
    pltpu.sync_copy(shared.at[:, pl.ds(ch, _CH)], red_v)

    def _reduce(j, carry):
        acc = red_v[0, pl.ds(j * 16, 16)]
        for k in range(1, _NS):
            acc = acc + red_v[k, pl.ds(j * 16, 16)]
        sum_v[pl.ds(j * 16, 16)] = acc
        return carry

    lax.fori_loop(0, _CH // 16, _reduce, 0)
    pltpu.sync_copy(sum_v.at[pl.ds(0, _CH)], sump_h.at[c, pl.ds(ch, _CH)])


@functools.partial(
    pl.kernel,
    out_type=jax.ShapeDtypeStruct((_NC, _NAGG, _D), jnp.float32),
    mesh=_mesh,
    compiler_params=pltpu.CompilerParams(needs_layout_passes=False),
    scratch_types=[
        pltpu.VMEM((_EPW,), jnp.int32),
        pltpu.VMEM((_EPW,), jnp.int32),
        pltpu.VMEM((_EPW,), jnp.float32),
        pltpu.VMEM((_NPAD,), jnp.float32),
        pltpu.VMEM((_CH,), jnp.float32),
        pltpu.VMEM((_G, _D), jnp.float32),
        pltpu.VMEM((_G, _D), jnp.float32),
        pltpu.VMEM((_G, _D), jnp.float32),
        pltpu.VMEM((_G, _D), jnp.float32),
        pltpu.VMEM_SHARED((_NAGG, _D), jnp.float32),
        pltpu.SemaphoreType.DMA,
        pltpu.SemaphoreType.DMA,
        pltpu.SemaphoreType.DMA,
        pltpu.SemaphoreType.DMA,
    ],
)
def _edge_phase_b(src_h, tgt_h, evals_h, sump_h, h_hbm, aggp_h,
                  src_v, tgt_v, w_v, recip_v, tmp_v,
                  rows0, rows1, scaled0, scaled1, agg_s,
                  gsem0, gsem1, ssem0, ssem1):
    c = lax.axis_index("c")
    s = lax.axis_index("s")
    wid = s * _NC + c
    base = wid * _EPW
    pltpu.sync_copy(src_h.at[pl.ds(base, _EPW)], src_v)
    pltpu.sync_copy(tgt_h.at[pl.ds(base, _EPW)], tgt_v)
    pltpu.sync_copy(evals_h.at[pl.ds(base, _EPW)], w_v)
    pltpu.sync_copy(sump_h.at[0], recip_v)

    def _recip_chunk(cidx, carry):
        cb = cidx * _CH
        pltpu.sync_copy(sump_h.at[1, pl.ds(cb, _CH)], tmp_v)

        def _recip(i, inner):
            o = cb + i * 16
            recip_v[pl.ds(o, 16)] = 1.0 / (recip_v[pl.ds(o, 16)]
                                           + tmp_v[pl.ds(i * 16, 16)] + 1e-10)
            return inner

        lax.fori_loop(0, _CH // 16, _recip, 0)
        return carry

    lax.fori_loop(0, _NPAD // _CH, _recip_chunk, 0)

    def _weights(i, carry):
        o = i * 16
        ti = tgt_v[pl.ds(o, 16)]
        w_v[pl.ds(o, 16)] = w_v[pl.ds(o, 16)] * plsc.load_gather(recip_v, [ti])
        return carry

    lax.fori_loop(0, _EPW // 16, _weights, 0)

    rows = (rows0, rows1)
    scaled = (scaled0, scaled1)
    gsem = (gsem0, gsem1)
    ssem = (ssem0, ssem1)

    zv = jnp.zeros((16,), jnp.float32)
    for buf in scaled:
        for j in range(_G):
            for k in range(_D // 16):
                buf[j, pl.ds(k * 16, 16)] = zv

    rbase = s * _CHA
    _ZF = _CHA // _G
    _ZR = _CHA - _ZF * _G

    def _zero(k, carry):
        pltpu.sync_copy(scaled0, agg_s.at[pl.ds(rbase + k * _G, _G), :])
        return carry

    lax.fori_loop(0, _ZF, _zero, 0)
    pltpu.sync_copy(scaled0.at[pl.ds(0, _ZR), :],
                    agg_s.at[pl.ds(rbase + _ZF * _G, _ZR), :])
    plsc.subcore_barrier()

    for b in (0, 1):
        si = src_v[pl.ds(b * _G, _G)]
        pltpu.async_copy(h_hbm.at[si], rows[b], gsem[b])
        ti = tgt_v[pl.ds(b * _G, _G)]
        pltpu.async_copy(scaled[b], agg_s.at[ti], ssem[b], add=True)

    def _outer(k2, carry):
        for b in (0, 1):
            g = k2 * 2 + b
            o = g * _G
            si = src_v[pl.ds(o, _G)]
            ti = tgt_v[pl.ds(o, _G)]
            pltpu.make_async_copy(h_hbm.at[si], rows[b], gsem[b]).wait()
            pltpu.make_async_copy(scaled[b], agg_s.at[ti], ssem[b]).wait()
            wv = w_v[pl.ds(o, _G)]
            for j in range(_G):
                wj = wv[j]
                for k in range(_D // 16):
                    sl = pl.ds(k * 16, 16)
                    scaled[b][j, sl] = rows[b][j, sl] * wj

            @pl.when(g + 2 < _NG)
            def _prefetch():
                si2 = src_v[pl.ds(o + 2 * _G, _G)]
                pltpu.async_copy(h_hbm.at[si2], rows[b], gsem[b])

            pltpu.async_copy(scaled[b], agg_s.at[ti], ssem[b], add=True)
        return carry

    lax.fori_loop(0, _NG2, _outer, 0)

    for b in (0, 1):
        o = (_NG - 2 + b) * _G
        ti = tgt_v[pl.ds(o, _G)]
        pltpu.make_async_copy(scaled[b], agg_s.at[ti], ssem[b]).wait()
    plsc.subcore_barrier()

    def _out(k, carry):
        r = rbase + k * _G
        pltpu.sync_copy(agg_s.at[pl.ds(r, _G), :], rows0)
        pltpu.sync_copy(rows0, aggp_h.at[c, pl.ds(r, _G), :])
        return carry

    lax.fori_loop(0, _ZF, _out, 0)
    r_tail = rbase + _ZF * _G
    pltpu.sync_copy(agg_s.at[pl.ds(r_tail, _ZR), :], rows0.at[pl.ds(0, _ZR), :])
    pltpu.sync_copy(rows0.at[pl.ds(0, _ZR), :], aggp_h.at[c, pl.ds(r_tail, _ZR), :])


def kernel(x, edge_index, W, b, Wm, bm, Wa):
    x_p = jnp.pad(x, ((0, _NPAD - _N), (0, 0)))
    h, s1c, s2c = _pre_call(x_p, W, b.reshape(1, _D), Wm, bm.reshape(1, 1),
                            Wa.reshape(2, _D))
    pad_e = _EPAD - _E
    src = jnp.concatenate([edge_index[0], jnp.zeros((pad_e,), jnp.int32)])
    tgt = jnp.concatenate([edge_index[1], jnp.full((pad_e,), _N, jnp.int32)])
    s1 = s1c.reshape(_NPAD)
    s2 = s2c.reshape(_NPAD)
    evals, sump = _edge_phase_a(src, tgt, s1, s2)
    aggp = _edge_phase_b(src, tgt, evals, sump, h)
    agg0 = jnp.pad(aggp[0], ((0, _NPAD - _NAGG), (0, 0)))
    agg1 = jnp.pad(aggp[1], ((0, _NPAD - _NAGG), (0, 0)))
    out = _post_call(h, agg0, agg1)
    return out[:_N]
